# Initial kernel scaffold; baseline (speedup 1.0000x reference)
import jax
import jax.numpy as jnp
from jax.experimental import pallas as pl


def kernel(spikes):
    raise NotImplementedError("write your pallas kernel here")


# speedup vs baseline: 25.7029x; 25.7029x over previous
import jax
import jax.numpy as jnp
from jax.experimental import pallas as pl

_TIME_STEP = 0.002
_SIZE = 100
_ROW_BLOCK = 32


def _onehot_kernel(s_ref, o_ref):
    bins = (s_ref[...] / _TIME_STEP).astype(jnp.int32)
    t = jax.lax.broadcasted_iota(jnp.int32, o_ref.shape, 1)
    o_ref[...] = (bins[:, None, :, :] == t).astype(jnp.float32)


def kernel(spikes):
    C, H, W = spikes.shape
    grid = (C, H // _ROW_BLOCK)
    return pl.pallas_call(
        _onehot_kernel,
        grid=grid,
        in_specs=[pl.BlockSpec((1, _ROW_BLOCK, W), lambda c, r: (c, r, 0))],
        out_specs=pl.BlockSpec((1, _SIZE, _ROW_BLOCK, W), lambda c, r: (c, 0, r, 0)),
        out_shape=jax.ShapeDtypeStruct((C, _SIZE, H, W), jnp.float32),
    )(spikes)


# speedup vs baseline: 27.8742x; 1.0845x over previous
import jax
import jax.numpy as jnp
from jax.experimental import pallas as pl

_TIME_STEP = 0.002
_SIZE = 100
_ROW_BLOCK = 64


def _onehot_kernel(s_ref, o_ref):
    bins = (s_ref[...] / _TIME_STEP).astype(jnp.int32)
    t = jax.lax.broadcasted_iota(jnp.int32, o_ref.shape, 1)
    o_ref[...] = (bins[:, None, :, :] == t).astype(jnp.float32)


def kernel(spikes):
    C, H, W = spikes.shape
    grid = (C, H // _ROW_BLOCK)
    return pl.pallas_call(
        _onehot_kernel,
        grid=grid,
        in_specs=[pl.BlockSpec((1, _ROW_BLOCK, W), lambda c, r: (c, r, 0))],
        out_specs=pl.BlockSpec((1, _SIZE, _ROW_BLOCK, W), lambda c, r: (c, 0, r, 0)),
        out_shape=jax.ShapeDtypeStruct((C, _SIZE, H, W), jnp.float32),
    )(spikes)
